# baseline (speedup 1.0000x reference)
import functools

import jax
import jax.numpy as jnp
from jax import lax
from jax.experimental import pallas as pl
from jax.experimental.pallas import tpu as pltpu
from jax.experimental.pallas import tpu_sc as plsc

_DIM = 128
_HALF = _DIM // 2
_NTOT = 4 * 8192

_info = plsc.get_sparse_core_info()
_NC, _NS = _info.num_cores, _info.num_subcores
_NW = _NC * _NS
_PER_W = _NTOT // _NW
_CHUNK = 128
_ROWS = _CHUNK // 2
_NCHUNK = 2 * _PER_W // _CHUNK
_NBUF = 4

_mesh = plsc.VectorSubcoreMesh(core_axis_name="c", subcore_axis_name="s")


@functools.partial(
    pl.kernel,
    out_type=(
        jax.ShapeDtypeStruct((2 * _NTOT, _HALF), jnp.float32),
        jax.ShapeDtypeStruct((2 * _NTOT, _HALF), jnp.float32),
    ),
    mesh=_mesh,
    compiler_params=pltpu.CompilerParams(use_tc_tiling_on_sc=False),
    scratch_types=(
        pltpu.VMEM((_NCHUNK, _CHUNK), jnp.int32),
        pltpu.VMEM((_NBUF, _CHUNK, _HALF), jnp.float32),
        pltpu.VMEM((_NBUF, _CHUNK, _HALF), jnp.float32),
        pltpu.SemaphoreType.DMA,
        pltpu.SemaphoreType.DMA,
    ),
)
def _rope_gather(idx_hbm, cos_hbm, sin_hbm, cos_out, sin_out,
                 idx_v, cos_v, sin_v, sem_g, sem_w):
    wid = lax.axis_index("s") * _NC + lax.axis_index("c")
    pltpu.sync_copy(idx_hbm.at[wid], idx_v)
    base = wid * 2 * _PER_W

    def gather(j):
        buf = j % _NBUF
        return (
            pltpu.async_copy(cos_hbm.at[idx_v.at[j]], cos_v.at[buf], sem_g),
            pltpu.async_copy(sin_hbm.at[idx_v.at[j]], sin_v.at[buf], sem_g),
        )

    def write(j):
        buf = j % _NBUF
        rows = pl.ds(base + j * _CHUNK, _CHUNK)
        return (
            pltpu.async_copy(cos_v.at[buf], cos_out.at[rows], sem_w),
            pltpu.async_copy(sin_v.at[buf], sin_out.at[rows], sem_w),
        )

    n_prime = min(_NBUF - 1, _NCHUNK)
    gathers = {j: gather(j) for j in range(n_prime)}
    writes = {}
    for j in range(_NCHUNK):
        ahead = j + _NBUF - 1
        if ahead < _NCHUNK:
            victim = ahead - _NBUF
            if victim >= 0:
                for w in writes.pop(victim):
                    w.wait()
            gathers[ahead] = gather(ahead)
        for g in gathers.pop(j):
            g.wait()
        writes[j] = write(j)
    for j in sorted(writes):
        for w in writes[j]:
            w.wait()


def kernel(position_ids, cos_cached, sin_cached):
    b, s = position_ids.shape
    idx = jnp.repeat(2 * position_ids.reshape(-1), 2).reshape(
        _NW, _NCHUNK, _CHUNK)
    cos_half = cos_cached.reshape(2 * cos_cached.shape[0], _HALF)
    sin_half = sin_cached.reshape(2 * sin_cached.shape[0], _HALF)
    cos2, sin2 = _rope_gather(idx, cos_half, sin_half)
    return (cos2.reshape(b, s, _DIM), sin2.reshape(b, s, _DIM))


# speedup vs baseline: 1.2523x; 1.2523x over previous
import functools

import jax
import jax.numpy as jnp
from jax import lax
from jax.experimental import pallas as pl
from jax.experimental.pallas import tpu as pltpu
from jax.experimental.pallas import tpu_sc as plsc

_DIM = 128
_HALF = _DIM // 2
_NTOT = 4 * 8192

_info = plsc.get_sparse_core_info()
_NC, _NS, _NL = _info.num_cores, _info.num_subcores, _info.num_lanes
_NW = _NC * _NS
_PER_W = _NTOT // _NW
_CHUNK = 128
_NCHUNK = _PER_W // _CHUNK
_NBUF = 6

_mesh = plsc.VectorSubcoreMesh(core_axis_name="c", subcore_axis_name="s")


@functools.partial(
    pl.kernel,
    out_type=(
        jax.ShapeDtypeStruct((_NTOT, 2, _HALF), jnp.float32),
        jax.ShapeDtypeStruct((_NTOT, 2, _HALF), jnp.float32),
    ),
    mesh=_mesh,
    compiler_params=pltpu.CompilerParams(use_tc_tiling_on_sc=False),
    scratch_types=(
        pltpu.VMEM((_NCHUNK, _CHUNK), jnp.int32),
        pltpu.VMEM((_NBUF, _CHUNK, _HALF), jnp.float32),
        pltpu.VMEM((_NBUF, _CHUNK, _HALF), jnp.float32),
        pltpu.SemaphoreType.DMA,
        pltpu.SemaphoreType.DMA,
    ),
)
def _rope_gather(idx_hbm, cos_hbm, sin_hbm, cos_out, sin_out,
                 idx_v, cos_v, sin_v, sem_g, sem_w):
    wid = lax.axis_index("s") * _NC + lax.axis_index("c")
    pltpu.sync_copy(idx_hbm.at[wid], idx_v)
    base = wid * _PER_W

    def double_ids(j):
        for c in range(_CHUNK // _NL):
            sl = pl.ds(c * _NL, _NL)
            idx_v[j, sl] = idx_v[j, sl] * 2

    def gather(j):
        buf = j % _NBUF
        return (
            pltpu.async_copy(cos_hbm.at[idx_v.at[j]], cos_v.at[buf], sem_g),
            pltpu.async_copy(sin_hbm.at[idx_v.at[j]], sin_v.at[buf], sem_g),
        )

    def write(j):
        buf = j % _NBUF
        rows = pl.ds(base + j * _CHUNK, _CHUNK)
        return (
            pltpu.async_copy(cos_v.at[buf], cos_out.at[rows, 0], sem_w),
            pltpu.async_copy(cos_v.at[buf], cos_out.at[rows, 1], sem_w),
            pltpu.async_copy(sin_v.at[buf], sin_out.at[rows, 0], sem_w),
            pltpu.async_copy(sin_v.at[buf], sin_out.at[rows, 1], sem_w),
        )

    n_prime = min(_NBUF - 1, _NCHUNK)
    for j in range(n_prime):
        double_ids(j)
    gathers = {j: gather(j) for j in range(n_prime)}
    writes = {}
    for j in range(_NCHUNK):
        ahead = j + _NBUF - 1
        if ahead < _NCHUNK:
            victim = ahead - _NBUF
            if victim >= 0:
                for w in writes.pop(victim):
                    w.wait()
            double_ids(ahead)
            gathers[ahead] = gather(ahead)
        for g in gathers.pop(j):
            g.wait()
        writes[j] = write(j)
    for j in sorted(writes):
        for w in writes[j]:
            w.wait()


def kernel(position_ids, cos_cached, sin_cached):
    b, s = position_ids.shape
    idx = position_ids.reshape(_NW, _NCHUNK, _CHUNK)
    cos_half = cos_cached.reshape(2 * cos_cached.shape[0], _HALF)
    sin_half = sin_cached.reshape(2 * sin_cached.shape[0], _HALF)
    cos3, sin3 = _rope_gather(idx, cos_half, sin_half)
    return (cos3.reshape(b, s, _DIM), sin3.reshape(b, s, _DIM))


# speedup vs baseline: 1.2536x; 1.0011x over previous
import functools

import jax
import jax.numpy as jnp
from jax import lax
from jax.experimental import pallas as pl
from jax.experimental.pallas import tpu as pltpu
from jax.experimental.pallas import tpu_sc as plsc

_DIM = 128
_HALF = _DIM // 2
_NTOT = 4 * 8192

_info = plsc.get_sparse_core_info()
_NC, _NS, _NL = _info.num_cores, _info.num_subcores, _info.num_lanes
_NW = _NC * _NS
_PER_W = _NTOT // _NW
_CHUNK = 128
_NCHUNK = _PER_W // _CHUNK
_NBUF = 4

_mesh = plsc.VectorSubcoreMesh(core_axis_name="c", subcore_axis_name="s")


@functools.partial(
    pl.kernel,
    out_type=(
        jax.ShapeDtypeStruct((_NTOT, 2, _HALF), jnp.float32),
        jax.ShapeDtypeStruct((_NTOT, 2, _HALF), jnp.float32),
    ),
    mesh=_mesh,
    compiler_params=pltpu.CompilerParams(use_tc_tiling_on_sc=False),
    scratch_types=(
        pltpu.VMEM((_NCHUNK, _CHUNK), jnp.int32),
        pltpu.VMEM((_NBUF, _CHUNK, _HALF), jnp.float32),
        pltpu.VMEM((_NBUF, _CHUNK, _HALF), jnp.float32),
        pltpu.SemaphoreType.DMA,
        pltpu.SemaphoreType.DMA,
    ),
)
def _rope_gather(idx_hbm, cos_hbm, sin_hbm, cos_out, sin_out,
                 idx_v, cos_v, sin_v, sem_g, sem_w):
    wid = lax.axis_index("s") * _NC + lax.axis_index("c")
    pltpu.sync_copy(idx_hbm.at[wid], idx_v)
    base = wid * _PER_W

    def double_ids(j):
        for c in range(_CHUNK // _NL):
            sl = pl.ds(c * _NL, _NL)
            idx_v[j, sl] = idx_v[j, sl] * 2

    def gather(j):
        buf = j % _NBUF
        return (
            pltpu.async_copy(cos_hbm.at[idx_v.at[j]], cos_v.at[buf], sem_g),
            pltpu.async_copy(sin_hbm.at[idx_v.at[j]], sin_v.at[buf], sem_g),
        )

    def write(j):
        buf = j % _NBUF
        rows = pl.ds(base + j * _CHUNK, _CHUNK)
        return (
            pltpu.async_copy(cos_v.at[buf], cos_out.at[rows, 0], sem_w),
            pltpu.async_copy(cos_v.at[buf], cos_out.at[rows, 1], sem_w),
            pltpu.async_copy(sin_v.at[buf], sin_out.at[rows, 0], sem_w),
            pltpu.async_copy(sin_v.at[buf], sin_out.at[rows, 1], sem_w),
        )

    n_prime = min(_NBUF - 1, _NCHUNK)
    for j in range(n_prime):
        double_ids(j)
    gathers = {j: gather(j) for j in range(n_prime)}
    writes = {}
    for j in range(_NCHUNK):
        ahead = j + _NBUF - 1
        if ahead < _NCHUNK:
            victim = ahead - _NBUF
            if victim >= 0:
                for w in writes.pop(victim):
                    w.wait()
            double_ids(ahead)
            gathers[ahead] = gather(ahead)
        for g in gathers.pop(j):
            g.wait()
        writes[j] = write(j)
    for j in sorted(writes):
        for w in writes[j]:
            w.wait()


def kernel(position_ids, cos_cached, sin_cached):
    b, s = position_ids.shape
    idx = position_ids.reshape(_NW, _NCHUNK, _CHUNK)
    cos_half = cos_cached.reshape(2 * cos_cached.shape[0], _HALF)
    sin_half = sin_cached.reshape(2 * sin_cached.shape[0], _HALF)
    cos3, sin3 = _rope_gather(idx, cos_half, sin_half)
    return (cos3.reshape(b, s, _DIM), sin3.reshape(b, s, _DIM))
